# baseline (speedup 1.0000x reference)
import functools

import jax
import jax.numpy as jnp
from jax import lax
from jax.experimental import pallas as pl
from jax.experimental.pallas import tpu as pltpu
from jax.experimental.pallas import tpu_sc as plsc

_N = 10000
_D = 128
_TOTAL = _N * _D
_CHUNK = 16000
_NCHUNKS = _TOTAL // _CHUNK
_NC = 2
_NS = 16
_NW = _NC * _NS
_LANES = 16


def _make_sc_add():
    mesh = plsc.VectorSubcoreMesh(core_axis_name="c", subcore_axis_name="s")

    @functools.partial(
        pl.kernel,
        mesh=mesh,
        out_type=jax.ShapeDtypeStruct((_TOTAL,), jnp.float32),
        scratch_types=[
            pltpu.VMEM((_CHUNK,), jnp.float32),
            pltpu.VMEM((_CHUNK,), jnp.float32),
        ],
    )
    def sc_add(x_hbm, pos_hbm, out_hbm, bufx, bufp):
        wid = lax.axis_index("s") * _NC + lax.axis_index("c")

        def do_chunk(c):
            base = c * _CHUNK
            pltpu.sync_copy(x_hbm.at[pl.ds(base, _CHUNK)], bufx)
            pltpu.sync_copy(pos_hbm.at[pl.ds(base, _CHUNK)], bufp)

            def body(i, _):
                sl = pl.ds(i * _LANES, _LANES)
                bufx[sl] = bufx[sl] + bufp[sl]
                return _

            lax.fori_loop(0, _CHUNK // _LANES, body, 0)
            pltpu.sync_copy(bufx, out_hbm.at[pl.ds(base, _CHUNK)])

        do_chunk(wid)
        do_chunk(wid + _NW)

        @pl.when(wid + 2 * _NW < _NCHUNKS)
        def _():
            do_chunk(wid + 2 * _NW)

    return sc_add


_sc_add = _make_sc_add()


def kernel(x, edge_index, pos_embedding):
    n, d = x.shape
    out_flat = _sc_add(x.reshape(-1), pos_embedding.reshape(-1))
    return out_flat.reshape(n, d)


# speedup vs baseline: 1.5853x; 1.5853x over previous
import functools

import jax
import jax.numpy as jnp
from jax import lax
from jax.experimental import pallas as pl
from jax.experimental.pallas import tpu as pltpu
from jax.experimental.pallas import tpu_sc as plsc

_N = 10000
_D = 128
_TOTAL = _N * _D
_CHUNK = 10000
_CPW = 4
_NC = 2
_NS = 16
_NW = _NC * _NS
_LANES = 16
_UNROLL = 5
_GROUPS = _CHUNK // (_LANES * _UNROLL)


def _make_sc_add():
    mesh = plsc.VectorSubcoreMesh(core_axis_name="c", subcore_axis_name="s")

    @functools.partial(
        pl.kernel,
        mesh=mesh,
        out_type=jax.ShapeDtypeStruct((_TOTAL,), jnp.float32),
        scratch_types=[
            pltpu.VMEM((_CHUNK,), jnp.float32),
            pltpu.VMEM((_CHUNK,), jnp.float32),
            pltpu.VMEM((_CHUNK,), jnp.float32),
            pltpu.VMEM((_CHUNK,), jnp.float32),
            pltpu.SemaphoreType.DMA,
            pltpu.SemaphoreType.DMA,
            pltpu.SemaphoreType.DMA,
            pltpu.SemaphoreType.DMA,
            pltpu.SemaphoreType.DMA,
            pltpu.SemaphoreType.DMA,
        ],
    )
    def sc_add(x_hbm, pos_hbm, out_hbm,
               bufx0, bufx1, bufp0, bufp1,
               sx0, sx1, sp0, sp1, so0, so1):
        wid = lax.axis_index("s") * _NC + lax.axis_index("c")
        bufx = (bufx0, bufx1)
        bufp = (bufp0, bufp1)
        sx = (sx0, sx1)
        sp = (sp0, sp1)
        so = (so0, so1)

        def start_in(t):
            b = t % 2
            base = (wid * _CPW + t) * _CHUNK
            hx = pltpu.async_copy(x_hbm.at[pl.ds(base, _CHUNK)], bufx[b], sx[b])
            hp = pltpu.async_copy(pos_hbm.at[pl.ds(base, _CHUNK)], bufp[b], sp[b])
            return hx, hp

        def start_out(t):
            b = t % 2
            base = (wid * _CPW + t) * _CHUNK
            return pltpu.async_copy(bufx[b], out_hbm.at[pl.ds(base, _CHUNK)], so[b])

        def compute(t):
            b = t % 2
            xv, pv = bufx[b], bufp[b]

            def body(i, _):
                e0 = i * (_LANES * _UNROLL)
                for j in range(_UNROLL):
                    sl = pl.ds(e0 + j * _LANES, _LANES)
                    xv[sl] = xv[sl] + pv[sl]
                return _

            lax.fori_loop(0, _GROUPS, body, 0)

        in_h = {0: start_in(0)}
        out_h = {}
        for t in range(_CPW):
            if t + 1 < _CPW:
                if t - 1 >= 0:
                    out_h[t - 1].wait()
                in_h[t + 1] = start_in(t + 1)
            hx, hp = in_h[t]
            hx.wait()
            hp.wait()
            compute(t)
            out_h[t] = start_out(t)
        out_h[_CPW - 2].wait()
        out_h[_CPW - 1].wait()

    return sc_add


_sc_add = _make_sc_add()


def kernel(x, edge_index, pos_embedding):
    n, d = x.shape
    out_flat = _sc_add(x.reshape(-1), pos_embedding.reshape(-1))
    return out_flat.reshape(n, d)
